# Initial kernel scaffold; baseline (speedup 1.0000x reference)
import jax
import jax.numpy as jnp
from jax.experimental import pallas as pl


def kernel(X):
    raise NotImplementedError("write your pallas kernel here")


# speedup vs baseline: 10.0980x; 10.0980x over previous
import jax
import jax.numpy as jnp
from jax.experimental import pallas as pl

LAM = 0.5


def _graphcut_body(x_ref, out_ref):
    x = x_ref[...]
    norm2 = jnp.sum(x * x, axis=1, keepdims=True)
    inv = jax.lax.rsqrt(norm2)
    xn = x * inv
    s = jnp.sum(xn, axis=0, keepdims=True)
    t = jnp.sum(xn * s, axis=1)
    d = norm2[:, 0] * (inv[:, 0] * inv[:, 0])
    out_ref[0, :] = t - LAM * d


def kernel(X):
    N, D = X.shape
    out = pl.pallas_call(
        _graphcut_body,
        out_shape=jax.ShapeDtypeStruct((1, N), X.dtype),
    )(X)
    return out[0]
